# baseline (speedup 1.0000x reference)
import functools

import jax
import jax.numpy as jnp
from jax import lax
from jax.experimental import pallas as pl
from jax.experimental.pallas import tpu as pltpu
from jax.experimental.pallas import tpu_sc as plsc

N = 20000
D_VF = 2048
NUM_CLASSES = 91
M = 50000
THRESHOLD = 0.5
SEL_W = 128


BLOCK_N = 1024


def _head_body(x_ref, w_ref, b_ref, logits_ref, probs_ref, sel_ref):
    lg = jnp.dot(x_ref[...], w_ref[...], preferred_element_type=jnp.float32)
    lg = lg + b_ref[...]
    probs = jax.nn.sigmoid(lg)
    logits_ref[...] = lg[:, :96].T
    probs_ref[...] = probs[:, :96].T
    sel_ref[...] = (probs > THRESHOLD).astype(jnp.int32)


def _region_head(x, W128, b128):
    grid = ((N + BLOCK_N - 1) // BLOCK_N,)
    return pl.pallas_call(
        _head_body,
        grid=grid,
        in_specs=[
            pl.BlockSpec((BLOCK_N, D_VF), lambda i: (i, 0)),
            pl.BlockSpec((D_VF, SEL_W), lambda i: (0, 0)),
            pl.BlockSpec((1, SEL_W), lambda i: (0, 0)),
        ],
        out_specs=[
            pl.BlockSpec((96, BLOCK_N), lambda i: (0, i)),
            pl.BlockSpec((96, BLOCK_N), lambda i: (0, i)),
            pl.BlockSpec((BLOCK_N, SEL_W), lambda i: (i, 0)),
        ],
        out_shape=[
            jax.ShapeDtypeStruct((96, N), jnp.float32),
            jax.ShapeDtypeStruct((96, N), jnp.float32),
            jax.ShapeDtypeStruct((N, SEL_W), jnp.int32),
        ],
    )(x, W128, b128)


NC = 2
NS = 16
NW = NC * NS
CHUNK = 128
N_CHUNKS = 13
BPW = CHUNK * N_CHUNKS
M_PAD = NW * BPW


def _idx_body(rows_ref, labels_ref, idx_ref):
    idx_ref[...] = rows_ref[...] * SEL_W + labels_ref[...]


def _flat_indices(rows_pad, labels_pad):
    return pl.pallas_call(
        _idx_body,
        out_shape=jax.ShapeDtypeStruct((M_PAD // CHUNK, CHUNK), jnp.int32),
    )(rows_pad, labels_pad)


def _gather_body(sel_hbm, idx_hbm, out_hbm, idx_v, gath_v, sem):
    wid = lax.axis_index("s") * NC + lax.axis_index("c")

    pltpu.sync_copy(idx_hbm.at[wid], idx_v)
    copies = []
    for c in range(N_CHUNKS):
        copies.append(
            pltpu.async_copy(sel_hbm.at[idx_v.at[c]], gath_v.at[c], sem))
    for cp in copies:
        cp.wait()
    pltpu.sync_copy(gath_v, out_hbm.at[wid])


def _box_masks(sel_flat, idx):
    mesh = plsc.VectorSubcoreMesh(core_axis_name="c", subcore_axis_name="s")
    f = pl.kernel(
        _gather_body,
        out_type=jax.ShapeDtypeStruct((NW, N_CHUNKS, CHUNK), jnp.int32),
        mesh=mesh,
        scratch_types=[
            pltpu.VMEM((N_CHUNKS, CHUNK), jnp.int32),
            pltpu.VMEM((N_CHUNKS, CHUNK), jnp.int32),
            pltpu.SemaphoreType.DMA,
        ],
    )
    return f(sel_flat, idx)


def kernel(x, boxes, box_labels, W, b):
    W128 = jnp.zeros((D_VF, SEL_W), jnp.float32).at[:, :NUM_CLASSES].set(W)
    b128 = jnp.zeros((1, SEL_W), jnp.float32).at[:, :NUM_CLASSES].set(
        b.reshape(1, NUM_CLASSES))
    logits_t, probs_t, sel = _region_head(x, W128, b128)

    rows_pad = jnp.zeros((M_PAD,), jnp.int32).at[:M].set(boxes[:, 0])
    labels_pad = jnp.zeros((M_PAD,), jnp.int32).at[:M].set(box_labels)
    rows_pad = rows_pad.reshape(M_PAD // CHUNK, CHUNK)
    labels_pad = labels_pad.reshape(M_PAD // CHUNK, CHUNK)

    idx = _flat_indices(rows_pad, labels_pad).reshape(NW, N_CHUNKS, CHUNK)
    mask_i32 = _box_masks(sel.reshape(-1), idx)
    box_masks = mask_i32.reshape(-1)[:M] != 0
    return (logits_t[:NUM_CLASSES].T, probs_t[:NUM_CLASSES].T, box_masks)


# speedup vs baseline: 1.1019x; 1.1019x over previous
import functools

import jax
import jax.numpy as jnp
from jax import lax
from jax.experimental import pallas as pl
from jax.experimental.pallas import tpu as pltpu
from jax.experimental.pallas import tpu_sc as plsc

N = 20000
D_VF = 2048
NUM_CLASSES = 91
M = 50000
THRESHOLD = 0.5
SEL_W = 128


BLOCK_N = 1024


def _head_body(x_ref, w_ref, b_ref, logits_ref, probs_ref, sel_ref):
    lg = jnp.dot(x_ref[...], w_ref[...], preferred_element_type=jnp.float32)
    lg = lg + b_ref[...]
    probs = jax.nn.sigmoid(lg)
    logits_ref[...] = lg[:, :96].T
    probs_ref[...] = probs[:, :96].T
    s = (probs > THRESHOLD).astype(jnp.int32)
    s3 = s.reshape(BLOCK_N // 2, 2, SEL_W)
    sel_ref[...] = s3[:, 0, :] | (s3[:, 1, :] << 16)


def _region_head(x, W128, b128):
    grid = ((N + BLOCK_N - 1) // BLOCK_N,)
    return pl.pallas_call(
        _head_body,
        grid=grid,
        in_specs=[
            pl.BlockSpec((BLOCK_N, D_VF), lambda i: (i, 0)),
            pl.BlockSpec((D_VF, SEL_W), lambda i: (0, 0)),
            pl.BlockSpec((1, SEL_W), lambda i: (0, 0)),
        ],
        out_specs=[
            pl.BlockSpec((96, BLOCK_N), lambda i: (0, i)),
            pl.BlockSpec((96, BLOCK_N), lambda i: (0, i)),
            pl.BlockSpec((BLOCK_N // 2, SEL_W), lambda i: (i, 0)),
        ],
        out_shape=[
            jax.ShapeDtypeStruct((96, N), jnp.float32),
            jax.ShapeDtypeStruct((96, N), jnp.float32),
            jax.ShapeDtypeStruct((N // 2, SEL_W), jnp.int32),
        ],
    )(x, W128, b128)


NC = 2
NS = 16
NW = NC * NS
CHUNK = 128
N_CHUNKS = 13
BPW = CHUNK * N_CHUNKS
M_PAD = NW * BPW
TABLE_WORDS = (N // 2) * SEL_W


def _idx_body(rows_ref, labels_ref, idx_ref, sh_ref):
    rows = rows_ref[...]
    idx_ref[...] = (rows >> 1) * SEL_W + labels_ref[...]
    sh_ref[...] = (rows & 1) * 16


def _flat_indices(rows_pad, labels_pad):
    return pl.pallas_call(
        _idx_body,
        out_shape=[
            jax.ShapeDtypeStruct((M_PAD // CHUNK, CHUNK), jnp.int32),
            jax.ShapeDtypeStruct((M_PAD // CHUNK, CHUNK), jnp.int32),
        ],
    )(rows_pad, labels_pad)


def _gather_body(sel_hbm, idx_hbm, sh_hbm, out_hbm,
                 table_sp, idx_v, sh_v, gath_v, sem):
    sid = lax.axis_index("s")
    wid = sid * NC + lax.axis_index("c")

    idx_cp = pltpu.async_copy(idx_hbm.at[wid], idx_v, sem)
    sh_cp = pltpu.async_copy(sh_hbm.at[wid], sh_v, sem)

    @pl.when(sid == 0)
    def _stage():
        pltpu.sync_copy(sel_hbm, table_sp)

    plsc.subcore_barrier()
    idx_cp.wait()
    sh_cp.wait()

    copies = []
    for c in range(N_CHUNKS):
        copies.append(
            pltpu.async_copy(table_sp.at[idx_v.at[c]], gath_v.at[c], sem))
    for cp in copies:
        cp.wait()

    for c in range(N_CHUNKS):
        for k in range(CHUNK // 16):
            w = gath_v[c, pl.ds(k * 16, 16)]
            sh = sh_v[c, pl.ds(k * 16, 16)]
            gath_v[c, pl.ds(k * 16, 16)] = (w >> sh) & 1

    pltpu.sync_copy(gath_v, out_hbm.at[wid])


def _box_masks(sel_flat, idx, sh):
    mesh = plsc.VectorSubcoreMesh(core_axis_name="c", subcore_axis_name="s")
    f = pl.kernel(
        _gather_body,
        out_type=jax.ShapeDtypeStruct((NW, N_CHUNKS, CHUNK), jnp.int32),
        mesh=mesh,
        scratch_types=[
            pltpu.VMEM_SHARED((TABLE_WORDS,), jnp.int32),
            pltpu.VMEM((N_CHUNKS, CHUNK), jnp.int32),
            pltpu.VMEM((N_CHUNKS, CHUNK), jnp.int32),
            pltpu.VMEM((N_CHUNKS, CHUNK), jnp.int32),
            pltpu.SemaphoreType.DMA,
        ],
    )
    return f(sel_flat, idx, sh)


def kernel(x, boxes, box_labels, W, b):
    W128 = jnp.zeros((D_VF, SEL_W), jnp.float32).at[:, :NUM_CLASSES].set(W)
    b128 = jnp.zeros((1, SEL_W), jnp.float32).at[:, :NUM_CLASSES].set(
        b.reshape(1, NUM_CLASSES))
    logits_t, probs_t, sel = _region_head(x, W128, b128)

    rows_pad = jnp.zeros((M_PAD,), jnp.int32).at[:M].set(boxes[:, 0])
    labels_pad = jnp.zeros((M_PAD,), jnp.int32).at[:M].set(box_labels)
    rows_pad = rows_pad.reshape(M_PAD // CHUNK, CHUNK)
    labels_pad = labels_pad.reshape(M_PAD // CHUNK, CHUNK)

    idx, sh = _flat_indices(rows_pad, labels_pad)
    idx = idx.reshape(NW, N_CHUNKS, CHUNK)
    sh = sh.reshape(NW, N_CHUNKS, CHUNK)
    mask_i32 = _box_masks(sel.reshape(-1), idx, sh)
    box_masks = mask_i32.reshape(-1)[:M] != 0
    return (logits_t[:NUM_CLASSES].T, probs_t[:NUM_CLASSES].T, box_masks)


# speedup vs baseline: 1.2540x; 1.1381x over previous
import functools

import jax
import jax.numpy as jnp
from jax import lax
from jax.experimental import pallas as pl
from jax.experimental.pallas import tpu as pltpu
from jax.experimental.pallas import tpu_sc as plsc

N = 20000
D_VF = 2048
NUM_CLASSES = 91
M = 50000
THRESHOLD = 0.5
SEL_W = 128


BLOCK_N = 1024
PACK = 32
N_PACKED = -(-N // PACK)


def _head_body(x_ref, w_ref, b_ref, logits_ref, probs_ref, sel_ref):
    lg = jnp.dot(x_ref[...], w_ref[...], preferred_element_type=jnp.float32)
    lg = lg + b_ref[...]
    probs = jax.nn.sigmoid(lg)
    logits_ref[...] = lg[:, :NUM_CLASSES].T
    probs_ref[...] = probs[:, :NUM_CLASSES].T
    s = (probs > THRESHOLD).astype(jnp.int32)
    s3 = s.reshape(BLOCK_N // PACK, PACK, SEL_W)
    w = s3[:, 0, :]
    for t in range(1, PACK):
        w = w | (s3[:, t, :] << t)
    sel_ref[...] = w


def _region_head(x, W128, b128):
    grid = ((N + BLOCK_N - 1) // BLOCK_N,)
    return pl.pallas_call(
        _head_body,
        grid=grid,
        in_specs=[
            pl.BlockSpec((BLOCK_N, D_VF), lambda i: (i, 0)),
            pl.BlockSpec((D_VF, SEL_W), lambda i: (0, 0)),
            pl.BlockSpec((1, SEL_W), lambda i: (0, 0)),
        ],
        out_specs=[
            pl.BlockSpec((NUM_CLASSES, BLOCK_N), lambda i: (0, i)),
            pl.BlockSpec((NUM_CLASSES, BLOCK_N), lambda i: (0, i)),
            pl.BlockSpec((BLOCK_N // PACK, SEL_W), lambda i: (i, 0)),
        ],
        out_shape=[
            jax.ShapeDtypeStruct((NUM_CLASSES, N), jnp.float32),
            jax.ShapeDtypeStruct((NUM_CLASSES, N), jnp.float32),
            jax.ShapeDtypeStruct((N_PACKED, SEL_W), jnp.int32),
        ],
    )(x, W128, b128)


NC = 2
NS = 16
NW = NC * NS
CHUNK = 128
N_CHUNKS = 13
BPW = CHUNK * N_CHUNKS
M_PAD = NW * BPW
TABLE_WORDS = N_PACKED * SEL_W


def _idx_body(rows_ref, labels_ref, idx_ref, sh_ref):
    rows = rows_ref[...]
    idx = (rows >> 5) * SEL_W + labels_ref[...]
    sh = rows & 31
    idx_ref[...] = idx.reshape(NW, N_CHUNKS, CHUNK)
    sh_ref[...] = sh.reshape(NW, N_CHUNKS, CHUNK)


def _flat_indices(rows_pad, labels_pad):
    return pl.pallas_call(
        _idx_body,
        out_shape=[
            jax.ShapeDtypeStruct((NW, N_CHUNKS, CHUNK), jnp.int32),
            jax.ShapeDtypeStruct((NW, N_CHUNKS, CHUNK), jnp.int32),
        ],
    )(rows_pad, labels_pad)


def _gather_body(sel_hbm, idx_hbm, sh_hbm, out_hbm,
                 table_sp, idx_v, sh_v, gath_v, sem):
    sid = lax.axis_index("s")
    wid = sid * NC + lax.axis_index("c")

    idx_cp = pltpu.async_copy(idx_hbm.at[wid], idx_v, sem)
    sh_cp = pltpu.async_copy(sh_hbm.at[wid], sh_v, sem)

    @pl.when(sid == 0)
    def _stage():
        pltpu.sync_copy(sel_hbm, table_sp)

    plsc.subcore_barrier()
    idx_cp.wait()
    sh_cp.wait()

    copies = []
    for c in range(N_CHUNKS):
        copies.append(
            pltpu.async_copy(table_sp.at[idx_v.at[c]], gath_v.at[c], sem))
    for cp in copies:
        cp.wait()

    for c in range(N_CHUNKS):
        for k in range(CHUNK // 16):
            w = gath_v[c, pl.ds(k * 16, 16)]
            sh = sh_v[c, pl.ds(k * 16, 16)]
            gath_v[c, pl.ds(k * 16, 16)] = (w >> sh) & 1

    pltpu.sync_copy(gath_v, out_hbm.at[wid])


def _box_masks(sel_flat, idx, sh):
    mesh = plsc.VectorSubcoreMesh(core_axis_name="c", subcore_axis_name="s")
    f = pl.kernel(
        _gather_body,
        out_type=jax.ShapeDtypeStruct((NW, N_CHUNKS, CHUNK), jnp.int32),
        mesh=mesh,
        scratch_types=[
            pltpu.VMEM_SHARED((TABLE_WORDS,), jnp.int32),
            pltpu.VMEM((N_CHUNKS, CHUNK), jnp.int32),
            pltpu.VMEM((N_CHUNKS, CHUNK), jnp.int32),
            pltpu.VMEM((N_CHUNKS, CHUNK), jnp.int32),
            pltpu.SemaphoreType.DMA,
        ],
    )
    return f(sel_flat, idx, sh)


def kernel(x, boxes, box_labels, W, b):
    W128 = jnp.zeros((D_VF, SEL_W), jnp.float32).at[:, :NUM_CLASSES].set(W)
    b128 = jnp.zeros((1, SEL_W), jnp.float32).at[:, :NUM_CLASSES].set(
        b.reshape(1, NUM_CLASSES))
    logits_t, probs_t, sel = _region_head(x, W128, b128)

    rows_pad = jnp.zeros((M_PAD,), jnp.int32).at[:M].set(boxes[:, 0])
    labels_pad = jnp.zeros((M_PAD,), jnp.int32).at[:M].set(box_labels)
    rows_pad = rows_pad.reshape(M_PAD // CHUNK, CHUNK)
    labels_pad = labels_pad.reshape(M_PAD // CHUNK, CHUNK)

    idx, sh = _flat_indices(rows_pad, labels_pad)
    mask_i32 = _box_masks(sel.reshape(-1), idx, sh)
    box_masks = mask_i32.reshape(-1)[:M] != 0
    return (logits_t.T, probs_t.T, box_masks)


# speedup vs baseline: 1.3054x; 1.0409x over previous
import functools

import jax
import jax.numpy as jnp
from jax import lax
from jax.experimental import pallas as pl
from jax.experimental.pallas import tpu as pltpu
from jax.experimental.pallas import tpu_sc as plsc

N = 20000
D_VF = 2048
NUM_CLASSES = 91
M = 50000
THRESHOLD = 0.5
SEL_W = 128


BLOCK_N = 1024
PACK = 32
N_PACKED = -(-N // PACK)


def _head_body(x_ref, w_ref, b_ref, logits_ref, probs_ref, sel_ref):
    pad = ((0, 0), (0, SEL_W - NUM_CLASSES))
    w128 = jnp.pad(w_ref[...], pad)
    b128 = jnp.pad(b_ref[...], pad)
    lg = jnp.dot(x_ref[...], w128, preferred_element_type=jnp.float32)
    lg = lg + b128
    probs = jax.nn.sigmoid(lg)
    logits_ref[...] = lg[:, :NUM_CLASSES].T
    probs_ref[...] = probs[:, :NUM_CLASSES].T
    s = (probs > THRESHOLD).astype(jnp.int32)
    s3 = s.reshape(BLOCK_N // PACK, PACK, SEL_W)
    w = s3[:, 0, :]
    for t in range(1, PACK):
        w = w | (s3[:, t, :] << t)
    sel_ref[...] = w


def _region_head(x, W, b2d):
    grid = ((N + BLOCK_N - 1) // BLOCK_N,)
    return pl.pallas_call(
        _head_body,
        grid=grid,
        in_specs=[
            pl.BlockSpec((BLOCK_N, D_VF), lambda i: (i, 0)),
            pl.BlockSpec((D_VF, NUM_CLASSES), lambda i: (0, 0)),
            pl.BlockSpec((1, NUM_CLASSES), lambda i: (0, 0)),
        ],
        out_specs=[
            pl.BlockSpec((NUM_CLASSES, BLOCK_N), lambda i: (0, i)),
            pl.BlockSpec((NUM_CLASSES, BLOCK_N), lambda i: (0, i)),
            pl.BlockSpec((BLOCK_N // PACK, SEL_W), lambda i: (i, 0)),
        ],
        out_shape=[
            jax.ShapeDtypeStruct((NUM_CLASSES, N), jnp.float32),
            jax.ShapeDtypeStruct((NUM_CLASSES, N), jnp.float32),
            jax.ShapeDtypeStruct((N_PACKED, SEL_W), jnp.int32),
        ],
    )(x, W, b2d)


NC = 2
NS = 16
NW = NC * NS
CHUNK = 128
N_CHUNKS = 13
BPW = CHUNK * N_CHUNKS
M_PAD = NW * BPW
TABLE_WORDS = N_PACKED * SEL_W


def _idx_body(rows_ref, labels_ref, idx_ref, sh_ref):
    rows = rows_ref[...]
    idx = (rows >> 5) * SEL_W + labels_ref[...]
    sh = rows & 31
    idx_ref[...] = idx.reshape(NW, N_CHUNKS, CHUNK)
    sh_ref[...] = sh.reshape(NW, N_CHUNKS, CHUNK)


def _flat_indices(rows_pad, labels_pad):
    return pl.pallas_call(
        _idx_body,
        out_shape=[
            jax.ShapeDtypeStruct((NW, N_CHUNKS, CHUNK), jnp.int32),
            jax.ShapeDtypeStruct((NW, N_CHUNKS, CHUNK), jnp.int32),
        ],
    )(rows_pad, labels_pad)


def _gather_body(sel_hbm, idx_hbm, sh_hbm, out_hbm,
                 table_sp, idx_v, sh_v, gath_v, sem):
    sid = lax.axis_index("s")
    wid = sid * NC + lax.axis_index("c")

    idx_cp = pltpu.async_copy(idx_hbm.at[wid], idx_v, sem)
    sh_cp = pltpu.async_copy(sh_hbm.at[wid], sh_v, sem)

    @pl.when(sid == 0)
    def _stage():
        pltpu.sync_copy(sel_hbm, table_sp)

    plsc.subcore_barrier()
    idx_cp.wait()
    sh_cp.wait()

    copies = []
    for c in range(N_CHUNKS):
        copies.append(
            pltpu.async_copy(table_sp.at[idx_v.at[c]], gath_v.at[c], sem))
    for cp in copies:
        cp.wait()

    for c in range(N_CHUNKS):
        for k in range(CHUNK // 16):
            w = gath_v[c, pl.ds(k * 16, 16)]
            sh = sh_v[c, pl.ds(k * 16, 16)]
            gath_v[c, pl.ds(k * 16, 16)] = (w >> sh) & 1

    pltpu.sync_copy(gath_v, out_hbm.at[wid])


def _box_masks(sel_flat, idx, sh):
    mesh = plsc.VectorSubcoreMesh(core_axis_name="c", subcore_axis_name="s")
    f = pl.kernel(
        _gather_body,
        out_type=jax.ShapeDtypeStruct((NW, N_CHUNKS, CHUNK), jnp.int32),
        mesh=mesh,
        scratch_types=[
            pltpu.VMEM_SHARED((TABLE_WORDS,), jnp.int32),
            pltpu.VMEM((N_CHUNKS, CHUNK), jnp.int32),
            pltpu.VMEM((N_CHUNKS, CHUNK), jnp.int32),
            pltpu.VMEM((N_CHUNKS, CHUNK), jnp.int32),
            pltpu.SemaphoreType.DMA,
        ],
    )
    return f(sel_flat, idx, sh)


def kernel(x, boxes, box_labels, W, b):
    logits_t, probs_t, sel = _region_head(x, W, b.reshape(1, NUM_CLASSES))

    rows_pad = jnp.zeros((M_PAD,), jnp.int32).at[:M].set(boxes[:, 0])
    labels_pad = jnp.zeros((M_PAD,), jnp.int32).at[:M].set(box_labels)
    rows_pad = rows_pad.reshape(M_PAD // CHUNK, CHUNK)
    labels_pad = labels_pad.reshape(M_PAD // CHUNK, CHUNK)

    idx, sh = _flat_indices(rows_pad, labels_pad)
    mask_i32 = _box_masks(sel.reshape(-1), idx, sh)
    box_masks = mask_i32.reshape(-1)[:M] != 0
    return (logits_t.T, probs_t.T, box_masks)


# speedup vs baseline: 1.3154x; 1.0077x over previous
import functools

import jax
import jax.numpy as jnp
from jax import lax
from jax.experimental import pallas as pl
from jax.experimental.pallas import tpu as pltpu
from jax.experimental.pallas import tpu_sc as plsc

N = 20000
D_VF = 2048
NUM_CLASSES = 91
M = 50000
THRESHOLD = 0.5
SEL_W = 128


BLOCK_N = 2048
PACK = 32
N_PACKED = -(-N // PACK)


def _head_body(x_ref, w_ref, b_ref, logits_ref, probs_ref, sel_ref):
    pad = ((0, 0), (0, SEL_W - NUM_CLASSES))
    w128 = jnp.pad(w_ref[...], pad)
    b128 = jnp.pad(b_ref[...], pad)
    lg = jnp.dot(x_ref[...], w128, preferred_element_type=jnp.float32)
    lg = lg + b128
    probs = jax.nn.sigmoid(lg)
    logits_ref[...] = lg[:, :NUM_CLASSES].T
    probs_ref[...] = probs[:, :NUM_CLASSES].T
    s = (probs > THRESHOLD).astype(jnp.int32)
    s3 = s.reshape(BLOCK_N // PACK, PACK, SEL_W)
    w = s3[:, 0, :]
    for t in range(1, PACK):
        w = w | (s3[:, t, :] << t)
    sel_ref[...] = w


def _region_head(x, W, b2d):
    grid = ((N + BLOCK_N - 1) // BLOCK_N,)
    return pl.pallas_call(
        _head_body,
        grid=grid,
        in_specs=[
            pl.BlockSpec((BLOCK_N, D_VF), lambda i: (i, 0)),
            pl.BlockSpec((D_VF, NUM_CLASSES), lambda i: (0, 0)),
            pl.BlockSpec((1, NUM_CLASSES), lambda i: (0, 0)),
        ],
        out_specs=[
            pl.BlockSpec((NUM_CLASSES, BLOCK_N), lambda i: (0, i)),
            pl.BlockSpec((NUM_CLASSES, BLOCK_N), lambda i: (0, i)),
            pl.BlockSpec((BLOCK_N // PACK, SEL_W), lambda i: (i, 0)),
        ],
        out_shape=[
            jax.ShapeDtypeStruct((NUM_CLASSES, N), jnp.float32),
            jax.ShapeDtypeStruct((NUM_CLASSES, N), jnp.float32),
            jax.ShapeDtypeStruct((N_PACKED, SEL_W), jnp.int32),
        ],
    )(x, W, b2d)


NC = 2
NS = 16
NW = NC * NS
CHUNK = 128
N_CHUNKS = 13
BPW = CHUNK * N_CHUNKS
M_PAD = NW * BPW
TABLE_WORDS = N_PACKED * SEL_W


def _idx_body(rows_ref, labels_ref, idx_ref, sh_ref):
    rows = rows_ref[...]
    idx = (rows >> 5) * SEL_W + labels_ref[...]
    sh = rows & 31
    idx_ref[...] = idx.reshape(NW, N_CHUNKS, CHUNK)
    sh_ref[...] = sh.reshape(NW, N_CHUNKS, CHUNK)


def _flat_indices(rows_pad, labels_pad):
    return pl.pallas_call(
        _idx_body,
        out_shape=[
            jax.ShapeDtypeStruct((NW, N_CHUNKS, CHUNK), jnp.int32),
            jax.ShapeDtypeStruct((NW, N_CHUNKS, CHUNK), jnp.int32),
        ],
    )(rows_pad, labels_pad)


def _gather_body(sel_hbm, idx_hbm, sh_hbm, out_hbm,
                 table_sp, idx_v, sh_v, gath_v, sem):
    sid = lax.axis_index("s")
    wid = sid * NC + lax.axis_index("c")

    idx_cp = pltpu.async_copy(idx_hbm.at[wid], idx_v, sem)
    sh_cp = pltpu.async_copy(sh_hbm.at[wid], sh_v, sem)

    @pl.when(sid == 0)
    def _stage():
        pltpu.sync_copy(sel_hbm, table_sp)

    plsc.subcore_barrier()
    idx_cp.wait()
    sh_cp.wait()

    copies = []
    for c in range(N_CHUNKS):
        copies.append(
            pltpu.async_copy(table_sp.at[idx_v.at[c]], gath_v.at[c], sem))
    for cp in copies:
        cp.wait()

    for c in range(N_CHUNKS):
        for k in range(CHUNK // 16):
            w = gath_v[c, pl.ds(k * 16, 16)]
            sh = sh_v[c, pl.ds(k * 16, 16)]
            gath_v[c, pl.ds(k * 16, 16)] = (w >> sh) & 1

    pltpu.sync_copy(gath_v, out_hbm.at[wid])


def _box_masks(sel_flat, idx, sh):
    mesh = plsc.VectorSubcoreMesh(core_axis_name="c", subcore_axis_name="s")
    f = pl.kernel(
        _gather_body,
        out_type=jax.ShapeDtypeStruct((NW, N_CHUNKS, CHUNK), jnp.int32),
        mesh=mesh,
        scratch_types=[
            pltpu.VMEM_SHARED((TABLE_WORDS,), jnp.int32),
            pltpu.VMEM((N_CHUNKS, CHUNK), jnp.int32),
            pltpu.VMEM((N_CHUNKS, CHUNK), jnp.int32),
            pltpu.VMEM((N_CHUNKS, CHUNK), jnp.int32),
            pltpu.SemaphoreType.DMA,
        ],
    )
    return f(sel_flat, idx, sh)


def kernel(x, boxes, box_labels, W, b):
    logits_t, probs_t, sel = _region_head(x, W, b.reshape(1, NUM_CLASSES))

    rows_pad = jnp.zeros((M_PAD,), jnp.int32).at[:M].set(boxes[:, 0])
    labels_pad = jnp.zeros((M_PAD,), jnp.int32).at[:M].set(box_labels)
    rows_pad = rows_pad.reshape(M_PAD // CHUNK, CHUNK)
    labels_pad = labels_pad.reshape(M_PAD // CHUNK, CHUNK)

    idx, sh = _flat_indices(rows_pad, labels_pad)
    mask_i32 = _box_masks(sel.reshape(-1), idx, sh)
    box_masks = mask_i32.reshape(-1)[:M] != 0
    return (logits_t.T, probs_t.T, box_masks)
